# Initial kernel scaffold; baseline (speedup 1.0000x reference)
import jax
import jax.numpy as jnp
from jax.experimental import pallas as pl


def kernel(x, edge_index, batch, W1_0, b1_0, W2_0, b2_0, eps_0, W1_1, b1_1, W2_1, b2_1, eps_1, W1_2, b1_2, W2_2, b2_2, eps_2):
    raise NotImplementedError("write your pallas kernel here")


# speedup vs baseline: 6.3558x; 6.3558x over previous
import functools

import jax
import jax.numpy as jnp
from jax import lax
from jax.experimental import pallas as pl
from jax.experimental.pallas import tpu as pltpu
from jax.experimental.pallas import tpu_sc as plsc

N = 10000
E = 320000
D = 128
G = 64

NC = 2
NS = 16
NW = NC * NS
EPW = E // NW
CH = 128
NFULL = EPW // CH
TAIL = EPW - NFULL * CH

@functools.cache
def _make_sc_aggregate():
    mesh = plsc.VectorSubcoreMesh(core_axis_name="c", subcore_axis_name="s")

    @functools.partial(
        pl.kernel,
        out_type=jax.ShapeDtypeStruct((NC, N, D), jnp.float32),
        mesh=mesh,
        scratch_types=[
            pltpu.VMEM((CH,), jnp.int32),
            pltpu.VMEM((CH,), jnp.int32),
            pltpu.VMEM((CH, D), jnp.float32),
            pltpu.VMEM((TAIL,), jnp.int32),
            pltpu.VMEM((TAIL,), jnp.int32),
            pltpu.VMEM((TAIL, D), jnp.float32),
            pltpu.VMEM_SHARED((N, D), jnp.float32),
            pltpu.SemaphoreType.DMA,
        ],
    )
    def _sc_aggregate(h_hbm, src_hbm, dst_hbm, zeros_hbm, out_hbm,
                      sidx, didx, rows, sidx_t, didx_t, rows_t, agg_sh, sem):
        cid = lax.axis_index("c")
        sid = lax.axis_index("s")
        wid = sid * NC + cid

        @pl.when(sid < 10)
        def _():
            pltpu.sync_copy(zeros_hbm.at[pl.ds(sid * 1000, 1000)],
                            agg_sh.at[pl.ds(sid * 1000, 1000)])

        plsc.subcore_barrier()

        def body(c, carry):
            base = wid * EPW + c * CH
            pltpu.sync_copy(src_hbm.at[pl.ds(base, CH)], sidx)
            pltpu.sync_copy(dst_hbm.at[pl.ds(base, CH)], didx)
            pltpu.async_copy(h_hbm.at[sidx], rows, sem).wait()
            pltpu.sync_copy(rows, agg_sh.at[didx], add=True)
            return carry

        lax.fori_loop(0, NFULL, body, 0)

        tbase = wid * EPW + NFULL * CH
        pltpu.sync_copy(src_hbm.at[pl.ds(tbase, TAIL)], sidx_t)
        pltpu.sync_copy(dst_hbm.at[pl.ds(tbase, TAIL)], didx_t)
        pltpu.async_copy(h_hbm.at[sidx_t], rows_t, sem).wait()
        pltpu.sync_copy(rows_t, agg_sh.at[didx_t], add=True)

        plsc.subcore_barrier()

        @pl.when(sid == 0)
        def _():
            pltpu.sync_copy(agg_sh, out_hbm.at[cid])

    return _sc_aggregate


BR = 1000
NBLK = N // BR


def _mlp_body(eps_ref, h_ref, p_ref, w1_ref, b1_ref, w2_ref, b2_ref, o_ref):
    eps = eps_ref[0, 0]
    u = (1.0 + eps) * h_ref[...] + p_ref[0] + p_ref[1]
    t = jnp.dot(u, w1_ref[...], preferred_element_type=jnp.float32)
    t = jnp.maximum(t + b1_ref[...], 0.0)
    o = jnp.dot(t, w2_ref[...], preferred_element_type=jnp.float32)
    o_ref[...] = jnp.maximum(o + b2_ref[...], 0.0)


def _mlp_final_body(eps_ref, h_ref, p_ref, w1_ref, b1_ref, w2_ref, b2_ref,
                    batch_ref, o_ref, r_ref):
    _mlp_body(eps_ref, h_ref, p_ref, w1_ref, b1_ref, w2_ref, b2_ref, o_ref)
    seg = lax.broadcasted_iota(jnp.int32, (G, BR), 0)
    onehot = (batch_ref[0] == seg).astype(jnp.float32)
    r = jnp.dot(onehot, o_ref[...], preferred_element_type=jnp.float32)
    step = pl.program_id(0)

    @pl.when(step == 0)
    def _():
        r_ref[...] = r

    @pl.when(step > 0)
    def _():
        r_ref[...] += r


def _tc_mlp(h, parts, w1, b1, w2, b2, eps):
    return pl.pallas_call(
        _mlp_body,
        grid=(NBLK,),
        in_specs=[
            pl.BlockSpec(memory_space=pltpu.SMEM),
            pl.BlockSpec((BR, D), lambda i: (i, 0)),
            pl.BlockSpec((NC, BR, D), lambda i: (0, i, 0)),
            pl.BlockSpec((D, D), lambda i: (0, 0)),
            pl.BlockSpec((D,), lambda i: (0,)),
            pl.BlockSpec((D, D), lambda i: (0, 0)),
            pl.BlockSpec((D,), lambda i: (0,)),
        ],
        out_specs=pl.BlockSpec((BR, D), lambda i: (i, 0)),
        out_shape=jax.ShapeDtypeStruct((N, D), jnp.float32),
    )(eps.reshape(1, 1), h, parts, w1, b1, w2, b2)


def _tc_mlp_final(h, parts, w1, b1, w2, b2, eps, batch3):
    return pl.pallas_call(
        _mlp_final_body,
        grid=(NBLK,),
        in_specs=[
            pl.BlockSpec(memory_space=pltpu.SMEM),
            pl.BlockSpec((BR, D), lambda i: (i, 0)),
            pl.BlockSpec((NC, BR, D), lambda i: (0, i, 0)),
            pl.BlockSpec((D, D), lambda i: (0, 0)),
            pl.BlockSpec((D,), lambda i: (0,)),
            pl.BlockSpec((D, D), lambda i: (0, 0)),
            pl.BlockSpec((D,), lambda i: (0,)),
            pl.BlockSpec((1, 1, BR), lambda i: (i, 0, 0)),
        ],
        out_specs=[
            pl.BlockSpec((BR, D), lambda i: (i, 0)),
            pl.BlockSpec((G, D), lambda i: (0, 0)),
        ],
        out_shape=[
            jax.ShapeDtypeStruct((N, D), jnp.float32),
            jax.ShapeDtypeStruct((G, D), jnp.float32),
        ],
    )(eps.reshape(1, 1), h, parts, w1, b1, w2, b2, batch3)


def kernel(x, edge_index, batch,
           W1_0, b1_0, W2_0, b2_0, eps_0,
           W1_1, b1_1, W2_1, b2_1, eps_1,
           W1_2, b1_2, W2_2, b2_2, eps_2):
    src = edge_index[0]
    dst = edge_index[1]
    zeros = jnp.zeros((N, D), jnp.float32)
    batch3 = batch.reshape(NBLK, 1, BR)
    params = [
        (W1_0, b1_0, W2_0, b2_0, eps_0),
        (W1_1, b1_1, W2_1, b2_1, eps_1),
        (W1_2, b1_2, W2_2, b2_2, eps_2),
    ]
    h = x
    for i, (w1, b1, w2, b2, eps) in enumerate(params):
        parts = _make_sc_aggregate()(h, src, dst, zeros)
        if i < 2:
            h = _tc_mlp(h, parts, w1, b1, w2, b2, eps)
        else:
            h, out = _tc_mlp_final(h, parts, w1, b1, w2, b2, eps, batch3)
    return out
